# baseline (speedup 1.0000x reference)
import functools

import jax
import jax.numpy as jnp
from jax import lax
from jax.experimental import pallas as pl
from jax.experimental.pallas import tpu as pltpu
from jax.experimental.pallas import tpu_sc as plsc

DIM = 32
BR = 8


def _make_kernel(nb: int, nt: int):
    info = plsc.get_sparse_core_info()
    nc, ns = info.num_cores, info.num_subcores
    nw = nc * ns
    rows_w = nb // nw
    n_chunks = rows_w // BR
    assert nb % nw == 0 and rows_w % BR == 0 and n_chunks % 2 == 0
    assert n_chunks >= 6
    splits = []
    off = 0
    while off < nt:
        w = min(128, nt - off)
        splits.append((off, w))
        off += w
    assert all(o % 8 == 0 for o, _ in splits)

    mesh = plsc.VectorSubcoreMesh(core_axis_name="c", subcore_axis_name="s")

    @functools.partial(
        pl.kernel,
        out_type=jax.ShapeDtypeStruct((nb, nt, DIM), jnp.float32),
        mesh=mesh,
        scratch_types=[
            pltpu.VMEM((2, BR, nt), jnp.int32),
            pltpu.VMEM((2, BR, nt, DIM), jnp.float32),
            pltpu.SemaphoreType.DMA,
            pltpu.SemaphoreType.DMA,
            pltpu.SemaphoreType.DMA,
            pltpu.SemaphoreType.DMA,
            pltpu.SemaphoreType.DMA,
            pltpu.SemaphoreType.DMA,
        ],
        compiler_params=pltpu.CompilerParams(use_tc_tiling_on_sc=False),
    )
    def gather_kernel(table_hbm, idx_hbm, out_hbm, idx_v, rows_v,
                      si0, sg0, sw0, si1, sg1, sw1):
        wid = lax.axis_index("s") * nc + lax.axis_index("c")
        row0 = wid * rows_w
        sem_i, sem_g, sem_w = (si0, si1), (sg0, sg1), (sw0, sw1)

        def fire_idx(c, b):
            pltpu.async_copy(
                idx_hbm.at[pl.ds(row0 + c * BR, BR)], idx_v.at[b], sem_i[b])

        def wait_idx(b):
            pltpu.make_async_copy(
                idx_hbm.at[pl.ds(row0, BR)], idx_v.at[b], sem_i[b]).wait()

        def fire_gathers(c, b):
            for j in range(BR):
                for off, w in splits:
                    pltpu.async_copy(
                        table_hbm.at[idx_v.at[b].at[j].at[pl.ds(off, w)]],
                        rows_v.at[b].at[j].at[pl.ds(off, w)],
                        sem_g[b])

        def wait_gathers(b):
            pltpu.make_async_copy(
                table_hbm.at[pl.ds(0, BR * nt)],
                rows_v.at[b], sem_g[b]).wait()

        def fire_wb(c, b):
            pltpu.async_copy(
                rows_v.at[b], out_hbm.at[pl.ds(row0 + c * BR, BR)], sem_w[b])

        def wait_wb(b):
            pltpu.make_async_copy(
                rows_v.at[b], out_hbm.at[pl.ds(row0, BR)], sem_w[b]).wait()

        def step(g, b, first, last):
            nb_ = 1 - b
            if not last or b == 0:
                wait_idx(nb_)
                if not first or b == 1:
                    wait_wb(nb_)
                fire_gathers(g + 1, nb_)
            wait_gathers(b)
            fire_wb(g, b)
            if not last:
                fire_idx(g + 2, b)

        fire_idx(0, 0)
        fire_idx(1, 1)
        wait_idx(0)
        fire_gathers(0, 0)

        step(0, 0, True, False)
        step(1, 1, True, False)

        def body(i, carry):
            step(2 * i, 0, False, False)
            step(2 * i + 1, 1, False, False)
            return carry

        lax.fori_loop(1, n_chunks // 2 - 1, body, 0)

        g_last = n_chunks - 2
        step(g_last, 0, False, True)
        step(g_last + 1, 1, False, True)

        wait_wb(0)
        wait_wb(1)

    return gather_kernel


def kernel(indices, table):
    nb, nt = indices.shape
    return _make_kernel(nb, nt)(table, indices.astype(jnp.int32))
